# baseline (speedup 1.0000x reference)
import functools

import jax
import jax.numpy as jnp
from jax import lax
from jax.experimental import pallas as pl
from jax.experimental.pallas import tpu as pltpu
from jax.experimental.pallas import tpu_sc as plsc

VOCAB_SIZE = 1000000
EMBED_DIM = 64
CONTEXT_LEN = 200
BATCH = 1024

_NUM_CORES = 2
_NUM_SUBCORES = 16
_NUM_WORKERS = _NUM_CORES * _NUM_SUBCORES
_ROWS_PER_WORKER = BATCH // _NUM_WORKERS

_mesh = plsc.VectorSubcoreMesh(core_axis_name="c", subcore_axis_name="s")


@functools.partial(
    pl.kernel,
    mesh=_mesh,
    compiler_params=pltpu.CompilerParams(use_tc_tiling_on_sc=False),
    out_type=jax.ShapeDtypeStruct((BATCH * CONTEXT_LEN, EMBED_DIM), jnp.float32),
    scratch_types=[
        pltpu.VMEM((CONTEXT_LEN,), jnp.int32),
        pltpu.VMEM((CONTEXT_LEN, EMBED_DIM), jnp.float32),
        pltpu.VMEM((CONTEXT_LEN, EMBED_DIM), jnp.float32),
        pltpu.SemaphoreType.DMA,
    ],
)
def _embed_kernel(idx_hbm, tok_hbm, pos_hbm, out_hbm, idx_v, rows_v, pos_v, sem):
    wid = lax.axis_index("s") * _NUM_CORES + lax.axis_index("c")
    pltpu.sync_copy(pos_hbm, pos_v)

    def row_body(r, carry):
        base = (wid * _ROWS_PER_WORKER + r) * CONTEXT_LEN
        pltpu.sync_copy(idx_hbm.at[pl.ds(base, CONTEXT_LEN)], idx_v)
        pltpu.async_copy(tok_hbm.at[idx_v], rows_v, sem).wait()

        def add_body(i, c):
            row = i // (EMBED_DIM // 16)
            col = (i % (EMBED_DIM // 16)) * 16
            rows_v[row, pl.ds(col, 16)] = (
                rows_v[row, pl.ds(col, 16)] + pos_v[row, pl.ds(col, 16)]
            )
            return c

        lax.fori_loop(0, CONTEXT_LEN * (EMBED_DIM // 16), add_body, 0)
        pltpu.sync_copy(rows_v, out_hbm.at[pl.ds(base, CONTEXT_LEN)])
        return carry

    lax.fori_loop(0, _ROWS_PER_WORKER, row_body, 0)


def kernel(inputs, token_table, position_table):
    idx = inputs.reshape(-1).astype(jnp.int32)
    out = _embed_kernel(idx, token_table, position_table)
    return out.reshape(BATCH, CONTEXT_LEN, EMBED_DIM)


# speedup vs baseline: 1.0279x; 1.0279x over previous
import functools

import jax
import jax.numpy as jnp
from jax import lax
from jax.experimental import pallas as pl
from jax.experimental.pallas import tpu as pltpu
from jax.experimental.pallas import tpu_sc as plsc

VOCAB_SIZE = 1000000
EMBED_DIM = 64
CONTEXT_LEN = 200
BATCH = 1024

_NUM_CORES = 2
_NUM_SUBCORES = 16
_NUM_WORKERS = _NUM_CORES * _NUM_SUBCORES
_BPW = BATCH // _NUM_WORKERS

_mesh = plsc.VectorSubcoreMesh(core_axis_name="c", subcore_axis_name="s")


@functools.partial(
    pl.kernel,
    mesh=_mesh,
    compiler_params=pltpu.CompilerParams(
        use_tc_tiling_on_sc=False, needs_layout_passes=False),
    out_type=jax.ShapeDtypeStruct((CONTEXT_LEN, 8, 8, 8, 128), jnp.float32),
    scratch_types=[
        pltpu.VMEM((CONTEXT_LEN, _BPW), jnp.int32),
        pltpu.VMEM((CONTEXT_LEN, EMBED_DIM), jnp.float32),
        pltpu.VMEM((_BPW, EMBED_DIM), jnp.float32),
        pltpu.VMEM((_BPW, EMBED_DIM), jnp.float32),
        pltpu.VMEM((EMBED_DIM, _BPW), jnp.float32),
        pltpu.VMEM((EMBED_DIM, _BPW), jnp.float32),
        pltpu.SemaphoreType.DMA,
        pltpu.SemaphoreType.DMA,
        pltpu.SemaphoreType.DMA,
        pltpu.SemaphoreType.DMA,
    ],
)
def _embed_kernel(idx_hbm, tok_hbm, pos_hbm, out_hbm,
                  idx_v, pos_v, rows0, rows1, blk0, blk1,
                  gs0, gs1, os0, os1):
    wid = lax.axis_index("s") * _NUM_CORES + lax.axis_index("c")
    b0 = wid * _BPW
    tc0 = b0 // 128
    c0 = b0 % 128

    pltpu.sync_copy(idx_hbm.at[:, pl.ds(b0, _BPW)], idx_v)
    pltpu.sync_copy(pos_hbm, pos_v)

    lane = lax.iota(jnp.int32, 16)

    def transpose_add(l, rows, blk):
        pv = [pos_v[l, pl.ds(dg * 16, 16)] for dg in range(4)]
        dv = [lane + dg * 16 for dg in range(4)]
        for j in range(_BPW):
            jv = jnp.full((16,), j, jnp.int32)
            for dg in range(4):
                x = rows[j, pl.ds(dg * 16, 16)] + pv[dg]
                plsc.store_scatter(blk, [dv[dg], jv], x)

    def out_tr(l, tr):
        return out_hbm.at[l, tr, tc0, :, pl.ds(c0, _BPW)]

    def start_out(l, blk, sem):
        for tr in range(8):
            pltpu.async_copy(blk.at[pl.ds(tr * 8, 8)], out_tr(l, tr), sem)

    def wait_out(l, blk, sem):
        for tr in range(8):
            pltpu.make_async_copy(blk.at[pl.ds(tr * 8, 8)], out_tr(l, tr), sem).wait()

    pltpu.async_copy(tok_hbm.at[idx_v.at[0]], rows0, gs0)

    def body(l2, carry):
        l0 = 2 * l2
        l1 = l0 + 1
        pltpu.async_copy(tok_hbm.at[idx_v.at[l1]], rows1, gs1)
        pltpu.make_async_copy(tok_hbm.at[idx_v.at[l0]], rows0, gs0).wait()

        @pl.when(l2 >= 1)
        def _():
            wait_out(l0, blk0, os0)

        transpose_add(l0, rows0, blk0)
        start_out(l0, blk0, os0)

        @pl.when(l2 < CONTEXT_LEN // 2 - 1)
        def _():
            pltpu.async_copy(tok_hbm.at[idx_v.at[l0 + 2]], rows0, gs0)

        pltpu.make_async_copy(tok_hbm.at[idx_v.at[l1]], rows1, gs1).wait()

        @pl.when(l2 >= 1)
        def _():
            wait_out(l1, blk1, os1)

        transpose_add(l1, rows1, blk1)
        start_out(l1, blk1, os1)
        return carry

    lax.fori_loop(0, CONTEXT_LEN // 2, body, 0)
    wait_out(CONTEXT_LEN - 2, blk0, os0)
    wait_out(CONTEXT_LEN - 1, blk1, os1)


def kernel(inputs, token_table, position_table):
    idx_t = jnp.transpose(inputs).astype(jnp.int32)
    out5 = _embed_kernel(idx_t, token_table, position_table)
    out = jnp.transpose(out5, (2, 4, 0, 1, 3)).reshape(BATCH, CONTEXT_LEN, EMBED_DIM)
    return out


# speedup vs baseline: 1.1331x; 1.1023x over previous
import functools

import jax
import jax.numpy as jnp
from jax import lax
from jax.experimental import pallas as pl
from jax.experimental.pallas import tpu as pltpu
from jax.experimental.pallas import tpu_sc as plsc

VOCAB_SIZE = 1000000
EMBED_DIM = 64
CONTEXT_LEN = 200
BATCH = 1024

_NUM_CORES = 2
_NUM_SUBCORES = 16
_NUM_WORKERS = _NUM_CORES * _NUM_SUBCORES
_BPW = BATCH // _NUM_WORKERS

_mesh = plsc.VectorSubcoreMesh(core_axis_name="c", subcore_axis_name="s")


@functools.partial(
    pl.kernel,
    mesh=_mesh,
    compiler_params=pltpu.CompilerParams(
        use_tc_tiling_on_sc=False, needs_layout_passes=False),
    out_type=jax.ShapeDtypeStruct((CONTEXT_LEN * BATCH, EMBED_DIM), jnp.float32),
    scratch_types=[
        pltpu.VMEM((CONTEXT_LEN, _BPW), jnp.int32),
        pltpu.VMEM((CONTEXT_LEN, EMBED_DIM), jnp.float32),
        pltpu.VMEM((_BPW, EMBED_DIM), jnp.float32),
        pltpu.VMEM((_BPW, EMBED_DIM), jnp.float32),
        pltpu.SemaphoreType.DMA,
        pltpu.SemaphoreType.DMA,
        pltpu.SemaphoreType.DMA,
        pltpu.SemaphoreType.DMA,
    ],
)
def _embed_kernel(idx_hbm, tok_hbm, pos_hbm, out_hbm,
                  idx_v, pos_v, rows0, rows1, gs0, gs1, os0, os1):
    wid = lax.axis_index("s") * _NUM_CORES + lax.axis_index("c")
    b0 = wid * _BPW

    pltpu.sync_copy(idx_hbm.at[:, pl.ds(b0, _BPW)], idx_v)
    pltpu.sync_copy(pos_hbm, pos_v)

    def add_pos(l, rows):
        pv = [pos_v[l, pl.ds(dg * 16, 16)] for dg in range(4)]
        for j in range(_BPW):
            for dg in range(4):
                rows[j, pl.ds(dg * 16, 16)] = rows[j, pl.ds(dg * 16, 16)] + pv[dg]

    def out_slice(l):
        return out_hbm.at[pl.ds(l * BATCH + b0, _BPW)]

    pltpu.async_copy(tok_hbm.at[idx_v.at[0]], rows0, gs0)

    def body(l2, carry):
        l0 = 2 * l2
        l1 = l0 + 1
        pltpu.async_copy(tok_hbm.at[idx_v.at[l1]], rows1, gs1)
        pltpu.make_async_copy(tok_hbm.at[idx_v.at[l0]], rows0, gs0).wait()

        @pl.when(l2 >= 1)
        def _():
            pltpu.make_async_copy(rows0, out_slice(l0), os0).wait()

        add_pos(l0, rows0)
        pltpu.async_copy(rows0, out_slice(l0), os0)

        @pl.when(l2 < CONTEXT_LEN // 2 - 1)
        def _():
            pltpu.async_copy(tok_hbm.at[idx_v.at[l0 + 2]], rows0, gs0)

        pltpu.make_async_copy(tok_hbm.at[idx_v.at[l1]], rows1, gs1).wait()

        @pl.when(l2 >= 1)
        def _():
            pltpu.make_async_copy(rows1, out_slice(l1), os1).wait()

        add_pos(l1, rows1)
        pltpu.async_copy(rows1, out_slice(l1), os1)
        return carry

    lax.fori_loop(0, CONTEXT_LEN // 2, body, 0)
    pltpu.make_async_copy(rows0, out_slice(CONTEXT_LEN - 2), os0).wait()
    pltpu.make_async_copy(rows1, out_slice(CONTEXT_LEN - 1), os1).wait()


def kernel(inputs, token_table, position_table):
    idx_t = jnp.transpose(inputs).astype(jnp.int32)
    out = _embed_kernel(idx_t, token_table, position_table)
    return jnp.transpose(out.reshape(CONTEXT_LEN, BATCH, EMBED_DIM), (1, 0, 2))
